# baseline (speedup 1.0000x reference)
import jax
import jax.numpy as jnp
from jax.experimental import pallas as pl
from jax.experimental.pallas import tpu as pltpu


_BW = 2304


_LOG2E = 1.4426950408889634


def _normalize_cols(qb):
    nrm = jnp.sqrt(jnp.sum(qb * qb, axis=0, keepdims=True))
    return qb * (_LOG2E / jnp.maximum(nrm, 1e-12))


def _pass1_kernel(q_ref, mem_ref, smmem_ref, upd_ref, csum_ref):
    j = pl.program_id(1)
    mem = mem_ref[:]
    qn = _normalize_cols(q_ref[0])
    s = jax.lax.dot_general(
        qn, mem, (((0,), (1,)), ((), ())),
        preferred_element_type=jnp.float32,
    )
    e = jnp.exp2(s)
    rsum = jnp.sum(e, axis=1, keepdims=True)
    rrec = 1.0 / rsum
    smmem_ref[:] = e * rrec
    eT = jax.lax.dot_general(
        mem, e, (((0,), (1,)), ((), ())),
        preferred_element_type=jnp.float32,
    )
    upd_ref[0] = eT * rrec.reshape(1, -1)

    @pl.when(j == 0)
    def _init():
        csum_ref[:] = jnp.zeros_like(csum_ref)

    csum_ref[0] += jnp.sum(e, axis=0, keepdims=True)


def _pass2_kernel(q_ref, mem_ref, csum_ref, smq_ref):
    qn = _normalize_cols(q_ref[0])
    s = jax.lax.dot_general(
        qn, mem_ref[:], (((0,), (1,)), ((), ())),
        preferred_element_type=jnp.float32,
    )
    csum = jnp.sum(csum_ref[:, 0, :], axis=0, keepdims=True)
    smq_ref[:] = jnp.exp2(s) * (1.0 / csum)


@jax.jit
def _memory_read(query_source, fusion_keys):
    b, d, h, w = query_source.shape
    m = fusion_keys.shape[0]
    hw = h * w
    n = b * hw
    jb = hw // _BW
    q = query_source.reshape(b, d, hw)

    smmem, upd, csum = pl.pallas_call(
        _pass1_kernel,
        grid=(b, jb),
        in_specs=[
            pl.BlockSpec((1, d, _BW), lambda bi, ji: (bi, 0, ji)),
            pl.BlockSpec((m, d), lambda bi, ji: (0, 0)),
        ],
        out_specs=[
            pl.BlockSpec((_BW, m), lambda bi, ji: (bi * jb + ji, 0)),
            pl.BlockSpec((1, d, _BW), lambda bi, ji: (bi, 0, ji)),
            pl.BlockSpec((1, 1, m), lambda bi, ji: (bi, 0, 0)),
        ],
        out_shape=[
            jax.ShapeDtypeStruct((n, m), jnp.float32),
            jax.ShapeDtypeStruct((b, d, hw), jnp.float32),
            jax.ShapeDtypeStruct((b, 1, m), jnp.float32),
        ],
        compiler_params=pltpu.CompilerParams(
            dimension_semantics=("parallel", "arbitrary"),
        ),
    )(q, fusion_keys)

    smq = pl.pallas_call(
        _pass2_kernel,
        grid=(b, jb),
        in_specs=[
            pl.BlockSpec((1, d, _BW), lambda bi, ji: (bi, 0, ji)),
            pl.BlockSpec((m, d), lambda bi, ji: (0, 0)),
            pl.BlockSpec((b, 1, m), lambda bi, ji: (0, 0, 0)),
        ],
        out_specs=pl.BlockSpec((_BW, m), lambda bi, ji: (bi * jb + ji, 0)),
        out_shape=jax.ShapeDtypeStruct((n, m), jnp.float32),
        compiler_params=pltpu.CompilerParams(
            dimension_semantics=("parallel", "parallel"),
        ),
    )(q, fusion_keys, csum)

    updated_query = upd.reshape(b, d, h, w)
    return updated_query, smq, smmem


def kernel(query_source, keys, only_update, fusion_keys):
    return _memory_read(query_source, fusion_keys)


# speedup vs baseline: 1.1519x; 1.1519x over previous
import jax
import jax.numpy as jnp
from jax.experimental import pallas as pl
from jax.experimental.pallas import tpu as pltpu


_HB = 16


_LOG2E = 1.4426950408889634


def _normalize_cols(qb):
    nrm = jnp.sqrt(jnp.sum(qb * qb, axis=0, keepdims=True))
    return qb * (_LOG2E / jnp.maximum(nrm, 1e-12))


def _pass1_kernel(q_ref, mem_ref, smmem_ref, upd_ref, csum_ref):
    j = pl.program_id(1)
    mem = mem_ref[:]
    d = q_ref.shape[1]
    qn = _normalize_cols(q_ref[0].reshape(d, -1))
    s = jax.lax.dot_general(
        qn, mem, (((0,), (1,)), ((), ())),
        preferred_element_type=jnp.float32,
    )
    e = jnp.exp2(s)
    rsum = jnp.sum(e, axis=1, keepdims=True)
    rrec = 1.0 / rsum
    smmem_ref[:] = e * rrec
    eT = jax.lax.dot_general(
        mem, e, (((0,), (1,)), ((), ())),
        preferred_element_type=jnp.float32,
    )
    upd_ref[0] = (eT * rrec.reshape(1, -1)).reshape(upd_ref.shape[1:])

    @pl.when(j == 0)
    def _init():
        csum_ref[:] = jnp.zeros_like(csum_ref)

    csum_ref[0] += jnp.sum(e, axis=0, keepdims=True)


def _pass2_kernel(q_ref, mem_ref, csum_ref, smq_ref):
    qn = _normalize_cols(q_ref[0].reshape(q_ref.shape[1], -1))
    s = jax.lax.dot_general(
        qn, mem_ref[:], (((0,), (1,)), ((), ())),
        preferred_element_type=jnp.float32,
    )
    csum = jnp.sum(csum_ref[:, 0, :], axis=0, keepdims=True)
    smq_ref[:] = jnp.exp2(s) * (1.0 / csum)


@jax.jit
def _memory_read(query_source, fusion_keys):
    b, d, h, w = query_source.shape
    m = fusion_keys.shape[0]
    hw = h * w
    n = b * hw
    jb = h // _HB
    bw = _HB * w
    q = query_source

    smmem, upd, csum = pl.pallas_call(
        _pass1_kernel,
        grid=(b, jb),
        in_specs=[
            pl.BlockSpec((1, d, _HB, w), lambda bi, ji: (bi, 0, ji, 0)),
            pl.BlockSpec((m, d), lambda bi, ji: (0, 0)),
        ],
        out_specs=[
            pl.BlockSpec((bw, m), lambda bi, ji: (bi * jb + ji, 0)),
            pl.BlockSpec((1, d, _HB, w), lambda bi, ji: (bi, 0, ji, 0)),
            pl.BlockSpec((1, 1, m), lambda bi, ji: (bi, 0, 0)),
        ],
        out_shape=[
            jax.ShapeDtypeStruct((n, m), jnp.float32),
            jax.ShapeDtypeStruct((b, d, h, w), jnp.float32),
            jax.ShapeDtypeStruct((b, 1, m), jnp.float32),
        ],
        compiler_params=pltpu.CompilerParams(
            dimension_semantics=("parallel", "arbitrary"),
        ),
    )(q, fusion_keys)

    smq = pl.pallas_call(
        _pass2_kernel,
        grid=(b, jb),
        in_specs=[
            pl.BlockSpec((1, d, _HB, w), lambda bi, ji: (bi, 0, ji, 0)),
            pl.BlockSpec((m, d), lambda bi, ji: (0, 0)),
            pl.BlockSpec((b, 1, m), lambda bi, ji: (0, 0, 0)),
        ],
        out_specs=pl.BlockSpec((bw, m), lambda bi, ji: (bi * jb + ji, 0)),
        out_shape=jax.ShapeDtypeStruct((n, m), jnp.float32),
        compiler_params=pltpu.CompilerParams(
            dimension_semantics=("parallel", "parallel"),
        ),
    )(q, fusion_keys, csum)

    updated_query = upd
    return updated_query, smq, smmem


def kernel(query_source, keys, only_update, fusion_keys):
    return _memory_read(query_source, fusion_keys)


# speedup vs baseline: 1.2045x; 1.0456x over previous
import jax
import jax.numpy as jnp
from jax.experimental import pallas as pl
from jax.experimental.pallas import tpu as pltpu


_HB = 16


_LOG2E = 1.4426950408889634


def _normalize_cols(qb):
    nrm = jnp.sqrt(jnp.sum(qb * qb, axis=0, keepdims=True))
    return qb * (_LOG2E / jnp.maximum(nrm, 1e-12))


def _pass1_kernel(q_ref, mem_ref, smmem_ref, csum_ref):
    j = pl.program_id(1)
    mem = mem_ref[:]
    d = q_ref.shape[1]
    qn = _normalize_cols(q_ref[0].reshape(d, -1))
    s = jax.lax.dot_general(
        qn, mem, (((0,), (1,)), ((), ())),
        preferred_element_type=jnp.float32,
    )
    e = jnp.exp2(s)
    rsum = jnp.sum(e, axis=1, keepdims=True)
    smmem_ref[:] = e * (1.0 / rsum)

    @pl.when(j == 0)
    def _init():
        csum_ref[:] = jnp.zeros_like(csum_ref)

    csum_ref[0] += jnp.sum(e, axis=0, keepdims=True)


def _pass2_kernel(q_ref, mem_ref, csum_ref, smq_ref, upd_ref):
    mem = mem_ref[:]
    qn = _normalize_cols(q_ref[0].reshape(q_ref.shape[1], -1))
    s = jax.lax.dot_general(
        qn, mem, (((0,), (1,)), ((), ())),
        preferred_element_type=jnp.float32,
    )
    e = jnp.exp2(s)
    csum = jnp.sum(csum_ref[:, 0, :], axis=0, keepdims=True)
    smq_ref[:] = e * (1.0 / csum)
    rrec = 1.0 / jnp.sum(e, axis=1, keepdims=True)
    eT = jax.lax.dot_general(
        mem, e, (((0,), (1,)), ((), ())),
        preferred_element_type=jnp.float32,
    )
    upd_ref[0] = (eT * rrec.reshape(1, -1)).reshape(upd_ref.shape[1:])


@jax.jit
def _memory_read(query_source, fusion_keys):
    b, d, h, w = query_source.shape
    m = fusion_keys.shape[0]
    hw = h * w
    n = b * hw
    jb = h // _HB
    bw = _HB * w
    q = query_source

    smmem, csum = pl.pallas_call(
        _pass1_kernel,
        grid=(b, jb),
        in_specs=[
            pl.BlockSpec((1, d, _HB, w), lambda bi, ji: (bi, 0, ji, 0)),
            pl.BlockSpec((m, d), lambda bi, ji: (0, 0)),
        ],
        out_specs=[
            pl.BlockSpec((bw, m), lambda bi, ji: (bi * jb + ji, 0)),
            pl.BlockSpec((1, 1, m), lambda bi, ji: (bi, 0, 0)),
        ],
        out_shape=[
            jax.ShapeDtypeStruct((n, m), jnp.float32),
            jax.ShapeDtypeStruct((b, 1, m), jnp.float32),
        ],
        compiler_params=pltpu.CompilerParams(
            dimension_semantics=("parallel", "arbitrary"),
        ),
    )(q, fusion_keys)

    smq, upd = pl.pallas_call(
        _pass2_kernel,
        grid=(b, jb),
        in_specs=[
            pl.BlockSpec((1, d, _HB, w), lambda bi, ji: (bi, 0, ji, 0)),
            pl.BlockSpec((m, d), lambda bi, ji: (0, 0)),
            pl.BlockSpec((b, 1, m), lambda bi, ji: (0, 0, 0)),
        ],
        out_specs=[
            pl.BlockSpec((bw, m), lambda bi, ji: (bi * jb + ji, 0)),
            pl.BlockSpec((1, d, _HB, w), lambda bi, ji: (bi, 0, ji, 0)),
        ],
        out_shape=[
            jax.ShapeDtypeStruct((n, m), jnp.float32),
            jax.ShapeDtypeStruct((b, d, h, w), jnp.float32),
        ],
        compiler_params=pltpu.CompilerParams(
            dimension_semantics=("parallel", "parallel"),
        ),
    )(q, fusion_keys, csum)

    updated_query = upd
    return updated_query, smq, smmem


def kernel(query_source, keys, only_update, fusion_keys):
    return _memory_read(query_source, fusion_keys)


# speedup vs baseline: 1.2416x; 1.0308x over previous
import jax
import jax.numpy as jnp
from jax.experimental import pallas as pl
from jax.experimental.pallas import tpu as pltpu


_HB = 16


_LOG2E = 1.4426950408889634


def _normalize_cols(qb):
    nrm = jnp.sqrt(jnp.sum(qb * qb, axis=0, keepdims=True))
    return qb * (_LOG2E / jnp.maximum(nrm, 1e-12))


def _pass1_kernel(q_ref, mem_ref, smmem_ref, csum_ref, qn_ref):
    j = pl.program_id(1)
    mem = mem_ref[:]
    d = q_ref.shape[1]
    qn = _normalize_cols(q_ref[0].reshape(d, -1))
    qn_ref[:] = qn
    s = jax.lax.dot_general(
        qn, mem, (((0,), (1,)), ((), ())),
        preferred_element_type=jnp.float32,
    )
    e = jnp.exp2(s)
    rsum = jnp.sum(e, axis=1, keepdims=True)
    smmem_ref[:] = e * (1.0 / rsum)

    @pl.when(j == 0)
    def _init():
        csum_ref[:] = jnp.zeros_like(csum_ref)

    csum_ref[0] += jnp.sum(e, axis=0, keepdims=True)


def _pass2_kernel(qn_ref, mem_ref, csum_ref, smq_ref, upd_ref):
    mem = mem_ref[:]
    qn = qn_ref[:]
    s = jax.lax.dot_general(
        qn, mem, (((0,), (1,)), ((), ())),
        preferred_element_type=jnp.float32,
    )
    e = jnp.exp2(s)
    csum = jnp.sum(csum_ref[:, 0, :], axis=0, keepdims=True)
    smq_ref[:] = e * (1.0 / csum)
    rrec = 1.0 / jnp.sum(e, axis=1, keepdims=True)
    eT = jax.lax.dot_general(
        mem, e, (((0,), (1,)), ((), ())),
        preferred_element_type=jnp.float32,
    )
    upd_ref[0] = (eT * rrec.reshape(1, -1)).reshape(upd_ref.shape[1:])


@jax.jit
def _memory_read(query_source, fusion_keys):
    b, d, h, w = query_source.shape
    m = fusion_keys.shape[0]
    hw = h * w
    n = b * hw
    jb = h // _HB
    bw = _HB * w
    q = query_source

    smmem, csum, qn2d = pl.pallas_call(
        _pass1_kernel,
        grid=(b, jb),
        in_specs=[
            pl.BlockSpec((1, d, _HB, w), lambda bi, ji: (bi, 0, ji, 0)),
            pl.BlockSpec((m, d), lambda bi, ji: (0, 0)),
        ],
        out_specs=[
            pl.BlockSpec((bw, m), lambda bi, ji: (bi * jb + ji, 0)),
            pl.BlockSpec((1, 1, m), lambda bi, ji: (bi, 0, 0)),
            pl.BlockSpec((d, bw), lambda bi, ji: (0, bi * jb + ji)),
        ],
        out_shape=[
            jax.ShapeDtypeStruct((n, m), jnp.float32),
            jax.ShapeDtypeStruct((b, 1, m), jnp.float32),
            jax.ShapeDtypeStruct((d, n), jnp.float32),
        ],
        compiler_params=pltpu.CompilerParams(
            dimension_semantics=("parallel", "arbitrary"),
        ),
    )(q, fusion_keys)

    smq, upd = pl.pallas_call(
        _pass2_kernel,
        grid=(b, jb),
        in_specs=[
            pl.BlockSpec((d, bw), lambda bi, ji: (0, bi * jb + ji)),
            pl.BlockSpec((m, d), lambda bi, ji: (0, 0)),
            pl.BlockSpec((b, 1, m), lambda bi, ji: (0, 0, 0)),
        ],
        out_specs=[
            pl.BlockSpec((bw, m), lambda bi, ji: (bi * jb + ji, 0)),
            pl.BlockSpec((1, d, _HB, w), lambda bi, ji: (bi, 0, ji, 0)),
        ],
        out_shape=[
            jax.ShapeDtypeStruct((n, m), jnp.float32),
            jax.ShapeDtypeStruct((b, d, h, w), jnp.float32),
        ],
        compiler_params=pltpu.CompilerParams(
            dimension_semantics=("parallel", "parallel"),
        ),
    )(qn2d, fusion_keys, csum)

    updated_query = upd
    return updated_query, smq, smmem


def kernel(query_source, keys, only_update, fusion_keys):
    return _memory_read(query_source, fusion_keys)
